# Initial kernel scaffold; baseline (speedup 1.0000x reference)
import jax
import jax.numpy as jnp
from jax.experimental import pallas as pl


def kernel(x, edge_index, pos_edge, neg_edge, W1, b1, W2, b2):
    raise NotImplementedError("write your pallas kernel here")


# speedup vs baseline: 5.2882x; 5.2882x over previous
import functools

import jax
import jax.numpy as jnp
from jax import lax
from jax.experimental import pallas as pl
from jax.experimental.pallas import tpu as pltpu
from jax.experimental.pallas import tpu_sc as plsc

N = 10000
NP = 10240
D = 128
OUT = 64
E = 320000
EDEC = 640000

NC = 2
NS = 16
NW = NC * NS
CH = 80
PER_W = E // NW
N_CHUNK = PER_W // CH
ROWS_W = NP // NS

NROW = NP // 128
HR_W = NROW // NS

DEC_PER_W = EDEC // NW
DEC_CHUNK = DEC_PER_W // CH

_mesh = plsc.VectorSubcoreMesh(
    core_axis_name="c", subcore_axis_name="s", num_cores=NC, num_subcores=NS)
_SC_PARAMS = pltpu.CompilerParams(needs_layout_passes=False)


HROW = NP * 8 // 128
DROW = NP // 128
DR_W = 8
DR_T = DROW // DR_W


@functools.partial(
    pl.kernel,
    out_type=jax.ShapeDtypeStruct((NC * DROW, 128), jnp.float32),
    mesh=_mesh,
    compiler_params=_SC_PARAMS,
    scratch_types=[
        pltpu.VMEM((PER_W,), jnp.int32),
        pltpu.VMEM((HROW, 128), jnp.float32),
        pltpu.VMEM((HROW,), jnp.int32),
        pltpu.VMEM((DR_W, 128), jnp.float32),
        pltpu.VMEM_SHARED((DROW, 128), jnp.float32),
    ],
)
def _deg_kernel(dst_hbm, out_hbm, didx, hist, rowids, stage, acc):
    c = lax.axis_index("c")
    s = lax.axis_index("s")
    wid = s * NC + c
    lanes = lax.broadcasted_iota(jnp.int32, (16,), 0)
    zv = jnp.zeros((16,), jnp.float32)
    ones = jnp.ones((16,), jnp.float32)
    lane_half = lanes & 7
    m_lo = lanes < 8
    m_hi = lanes >= 8

    for r in range(DR_W):
        for j in range(128 // 16):
            stage[r, pl.ds(j * 16, 16)] = zv

    @pl.when(s < DR_T)
    def _():
        pltpu.sync_copy(stage, acc.at[pl.ds(s * DR_W, DR_W), :])

    def zbody(r, carry):
        for j in range(128 // 16):
            hist[r, pl.ds(j * 16, 16)] = zv
        return carry

    lax.fori_loop(0, HROW, zbody, 0)

    def rbody(k, carry):
        rv = lanes + k * 16
        rowids[pl.ds(k * 16, 16)] = rv - (rv // DROW) * DROW
        return carry

    lax.fori_loop(0, HROW // 16, rbody, 0)

    pltpu.sync_copy(dst_hbm.at[pl.ds(wid * PER_W, PER_W)], didx)

    def ebody(j, carry):
        v = didx[pl.ds(j * 16, 16)]
        a = v + lane_half * NP
        r = a >> 7
        cc = a & 127
        plsc.addupdate_scatter(hist, [r, cc], ones, mask=m_lo)
        plsc.addupdate_scatter(hist, [r, cc], ones, mask=m_hi)
        return carry

    lax.fori_loop(0, PER_W // 16, ebody, 0)
    plsc.subcore_barrier()
    pltpu.sync_copy(hist, acc.at[rowids], add=True)
    plsc.subcore_barrier()

    @pl.when(s < DR_T)
    def _():
        pltpu.sync_copy(acc.at[pl.ds(s * DR_W, DR_W), :], stage)
        pltpu.sync_copy(stage, out_hbm.at[pl.ds(c * DROW + s * DR_W, DR_W), :])


@functools.partial(
    pl.kernel,
    out_type=jax.ShapeDtypeStruct((NC * NP, D), jnp.float32),
    mesh=_mesh,
    compiler_params=_SC_PARAMS,
    scratch_types=[
        pltpu.VMEM((CH,), jnp.int32),
        pltpu.VMEM((CH,), jnp.int32),
        pltpu.VMEM((CH, D), jnp.float32),
        pltpu.VMEM_SHARED((NP, D), jnp.float32),
    ],
)
def _scatter_kernel(g_hbm, src_hbm, dst_hbm, out_hbm, sidx, didx, rows, acc):
    c = lax.axis_index("c")
    s = lax.axis_index("s")
    wid = s * NC + c
    zv = jnp.zeros((16,), jnp.float32)

    def zbody(r, carry):
        for j in range(D // 16):
            rows[r, pl.ds(j * 16, 16)] = zv
        return carry

    lax.fori_loop(0, CH, zbody, 0)
    for j in range(ROWS_W // CH):
        pltpu.sync_copy(rows, acc.at[pl.ds(s * ROWS_W + j * CH, CH), :])
    plsc.subcore_barrier()

    def ebody(i, carry):
        base = wid * PER_W + i * CH
        pltpu.sync_copy(src_hbm.at[pl.ds(base, CH)], sidx)
        pltpu.sync_copy(dst_hbm.at[pl.ds(base, CH)], didx)
        pltpu.sync_copy(g_hbm.at[sidx], rows)
        pltpu.sync_copy(rows, acc.at[didx], add=True)
        return carry

    lax.fori_loop(0, N_CHUNK, ebody, 0)
    plsc.subcore_barrier()

    for j in range(ROWS_W // CH):
        r0 = s * ROWS_W + j * CH
        pltpu.sync_copy(acc.at[pl.ds(r0, CH), :], rows)
        pltpu.sync_copy(rows, out_hbm.at[pl.ds(c * NP + r0, CH), :])


@functools.partial(
    pl.kernel,
    out_type=jax.ShapeDtypeStruct((EDEC,), jnp.float32),
    mesh=_mesh,
    compiler_params=_SC_PARAMS,
    scratch_types=[
        pltpu.VMEM((CH,), jnp.int32),
        pltpu.VMEM((CH,), jnp.int32),
        pltpu.VMEM((CH, D), jnp.float32),
        pltpu.VMEM((CH, D), jnp.float32),
        pltpu.VMEM((DEC_PER_W,), jnp.float32),
    ],
)
def _decoder_kernel(lat_hbm, ia_hbm, ib_hbm, out_hbm, ia, ib, ra, rb, obuf):
    c = lax.axis_index("c")
    s = lax.axis_index("s")
    wid = s * NC + c
    lanes = lax.broadcasted_iota(jnp.int32, (16,), 0)

    def ebody(i, carry):
        base = wid * DEC_PER_W + i * CH
        pltpu.sync_copy(ia_hbm.at[pl.ds(base, CH)], ia)
        pltpu.sync_copy(ib_hbm.at[pl.ds(base, CH)], ib)
        pltpu.sync_copy(lat_hbm.at[ia], ra)
        pltpu.sync_copy(lat_hbm.at[ib], rb)
        for g in range(CH // 16):
            evec = lanes + g * 16
            acc = jnp.zeros((16,), jnp.float32)
            for k in range(OUT):
                kvec = jnp.full((16,), k, jnp.int32)
                va = plsc.load_gather(ra, [evec, kvec])
                vb = plsc.load_gather(rb, [evec, kvec])
                acc = acc + va * vb
            obuf[pl.ds(i * CH + g * 16, 16)] = acc
        return carry

    lax.fori_loop(0, DEC_CHUNK, ebody, 0)
    pltpu.sync_copy(obuf, out_hbm.at[pl.ds(wid * DEC_PER_W, DEC_PER_W)])


RB = 2048
GRID = NP // RB


def _dinv_block(degp):
    deg = degp[0] + degp[1] + 1.0
    return 1.0 / jnp.sqrt(deg)


def _tc1_body(x_ref, w1_ref, degp_ref, g1_ref):
    dinv = _dinv_block(degp_ref[...])
    h = jnp.dot(x_ref[...], w1_ref[...], preferred_element_type=jnp.float32)
    g1_ref[...] = h * dinv


def _tc2_body(p_ref, g1_ref, degp_ref, b1_ref, w2_ref, g2_ref):
    dinv = _dinv_block(degp_ref[...])
    agg = (p_ref[0] + p_ref[1] + g1_ref[...]) * dinv + b1_ref[...]
    h2 = jnp.maximum(agg, 0.0)
    t = jnp.dot(h2, w2_ref[...], preferred_element_type=jnp.float32) * dinv
    g2_ref[...] = jnp.concatenate(
        [t, jnp.zeros((RB, D - OUT), jnp.float32)], axis=1)


def _tc3_body(q_ref, g2_ref, degp_ref, b2_ref, lat_ref):
    dinv = _dinv_block(degp_ref[...])
    qsum = (q_ref[0] + q_ref[1] + g2_ref[...])[:, :OUT]
    lat = qsum * dinv + b2_ref[...]
    lat_ref[...] = jnp.concatenate(
        [lat, jnp.zeros((RB, D - OUT), jnp.float32)], axis=1)


def _row_spec(w):
    return pl.BlockSpec((RB, w), lambda i: (i, 0))


def _pair_spec(w):
    return pl.BlockSpec((2, RB, w), lambda i: (0, i, 0))


_DEGP_SPEC = pl.BlockSpec((2, RB, 1), lambda i: (0, i, 0))


def _full_spec(a, b):
    return pl.BlockSpec((a, b), lambda i: (0, 0))


_tc1 = pl.pallas_call(
    _tc1_body,
    grid=(GRID,),
    in_specs=[_row_spec(D), _full_spec(D, D), _DEGP_SPEC],
    out_specs=_row_spec(D),
    out_shape=jax.ShapeDtypeStruct((NP, D), jnp.float32),
)

_tc2 = pl.pallas_call(
    _tc2_body,
    grid=(GRID,),
    in_specs=[_pair_spec(D), _row_spec(D), _DEGP_SPEC,
              _full_spec(1, D), _full_spec(D, OUT)],
    out_specs=_row_spec(D),
    out_shape=jax.ShapeDtypeStruct((NP, D), jnp.float32),
)

_tc3 = pl.pallas_call(
    _tc3_body,
    grid=(GRID,),
    in_specs=[_pair_spec(D), _row_spec(D), _DEGP_SPEC,
              _full_spec(1, OUT)],
    out_specs=_row_spec(D),
    out_shape=jax.ShapeDtypeStruct((NP, D), jnp.float32),
)


def kernel(x, edge_index, pos_edge, neg_edge, W1, b1, W2, b2):
    src = edge_index[0]
    dst = edge_index[1]
    x_pad = jnp.pad(x, ((0, NP - N), (0, 0)))

    degp = _deg_kernel(dst).reshape(2, NP, 1)
    g1 = _tc1(x_pad, W1, degp)
    p = _scatter_kernel(g1, src, dst).reshape(2, NP, D)
    g2 = _tc2(p, g1, degp, b1.reshape(1, D), W2)
    q = _scatter_kernel(g2, src, dst).reshape(2, NP, D)
    latent = _tc3(q, g2, degp, b2.reshape(1, OUT))

    ia = jnp.concatenate([pos_edge[0], neg_edge[0]])
    ib = jnp.concatenate([pos_edge[1], neg_edge[1]])
    return _decoder_kernel(latent, ia, ib)


# speedup vs baseline: 7.8218x; 1.4791x over previous
import functools

import jax
import jax.numpy as jnp
from jax import lax
from jax.experimental import pallas as pl
from jax.experimental.pallas import tpu as pltpu
from jax.experimental.pallas import tpu_sc as plsc

N = 10000
NP = 10240
D = 128
OUT = 64
E = 320000
EDEC = 640000

NC = 2
NS = 16
NW = NC * NS
CH = 80
PER_W = E // NW
N_CHUNK = PER_W // CH
ROWS_W = NP // NS

NROW = NP // 128
HR_W = NROW // NS

DEC_PER_W = EDEC // NW
DEC_CHUNK = DEC_PER_W // CH

_mesh = plsc.VectorSubcoreMesh(
    core_axis_name="c", subcore_axis_name="s", num_cores=NC, num_subcores=NS)
_SC_PARAMS = pltpu.CompilerParams(needs_layout_passes=False)


HROW = NP * 8 // 128
DROW = NP // 128
DR_W = 8
DR_T = DROW // DR_W


@functools.partial(
    pl.kernel,
    out_type=jax.ShapeDtypeStruct((NC * DROW, 128), jnp.float32),
    mesh=_mesh,
    compiler_params=_SC_PARAMS,
    scratch_types=[
        pltpu.VMEM((PER_W,), jnp.int32),
        pltpu.VMEM((HROW, 128), jnp.float32),
        pltpu.VMEM((HROW,), jnp.int32),
        pltpu.VMEM((DR_W, 128), jnp.float32),
        pltpu.VMEM_SHARED((DROW, 128), jnp.float32),
    ],
)
def _deg_kernel(dst_hbm, out_hbm, didx, hist, rowids, stage, acc):
    c = lax.axis_index("c")
    s = lax.axis_index("s")
    wid = s * NC + c
    lanes = lax.broadcasted_iota(jnp.int32, (16,), 0)
    zv = jnp.zeros((16,), jnp.float32)
    ones = jnp.ones((16,), jnp.float32)
    lane_half = lanes & 7
    m_lo = lanes < 8
    m_hi = lanes >= 8

    for r in range(DR_W):
        for j in range(128 // 16):
            stage[r, pl.ds(j * 16, 16)] = zv

    @pl.when(s < DR_T)
    def _():
        pltpu.sync_copy(stage, acc.at[pl.ds(s * DR_W, DR_W), :])

    def zbody(r, carry):
        for j in range(128 // 16):
            hist[r, pl.ds(j * 16, 16)] = zv
        return carry

    lax.fori_loop(0, HROW, zbody, 0)

    def rbody(k, carry):
        rv = lanes + k * 16
        rowids[pl.ds(k * 16, 16)] = rv - (rv // DROW) * DROW
        return carry

    lax.fori_loop(0, HROW // 16, rbody, 0)

    pltpu.sync_copy(dst_hbm.at[pl.ds(wid * PER_W, PER_W)], didx)

    def ebody(j, carry):
        v = didx[pl.ds(j * 16, 16)]
        a = v + lane_half * NP
        r = a >> 7
        cc = a & 127
        plsc.addupdate_scatter(hist, [r, cc], ones, mask=m_lo)
        plsc.addupdate_scatter(hist, [r, cc], ones, mask=m_hi)
        return carry

    lax.fori_loop(0, PER_W // 16, ebody, 0)
    plsc.subcore_barrier()
    pltpu.sync_copy(hist, acc.at[rowids], add=True)
    plsc.subcore_barrier()

    @pl.when(s < DR_T)
    def _():
        pltpu.sync_copy(acc.at[pl.ds(s * DR_W, DR_W), :], stage)
        pltpu.sync_copy(stage, out_hbm.at[pl.ds(c * DROW + s * DR_W, DR_W), :])


LASTC = N_CHUNK - 1


@functools.partial(
    pl.kernel,
    out_type=jax.ShapeDtypeStruct((NC * NP, D), jnp.float32),
    mesh=_mesh,
    compiler_params=_SC_PARAMS,
    scratch_types=[
        pltpu.VMEM((CH,), jnp.int32), pltpu.VMEM((CH,), jnp.int32),
        pltpu.VMEM((CH,), jnp.int32), pltpu.VMEM((CH,), jnp.int32),
        pltpu.VMEM((CH, D), jnp.float32), pltpu.VMEM((CH, D), jnp.float32),
        pltpu.VMEM_SHARED((NP, D), jnp.float32),
        pltpu.SemaphoreType.DMA, pltpu.SemaphoreType.DMA,
        pltpu.SemaphoreType.DMA, pltpu.SemaphoreType.DMA,
        pltpu.SemaphoreType.DMA, pltpu.SemaphoreType.DMA,
    ],
)
def _scatter_kernel(g_hbm, src_hbm, dst_hbm, out_hbm,
                    s0, s1, d0, d1, r0, r1, acc,
                    is0, is1, gs0, gs1, ws0, ws1):
    c = lax.axis_index("c")
    s = lax.axis_index("s")
    wid = s * NC + c
    S = [s0, s1]
    Dx = [d0, d1]
    R = [r0, r1]
    IS = [is0, is1]
    GS = [gs0, gs1]
    WS = [ws0, ws1]
    zv = jnp.zeros((16,), jnp.float32)

    def issue_idx(i, b):
        base = wid * PER_W + i * CH
        pltpu.async_copy(src_hbm.at[pl.ds(base, CH)], S[b], IS[b])
        pltpu.async_copy(dst_hbm.at[pl.ds(base, CH)], Dx[b], IS[b])

    def wait_idx(b):
        pltpu.make_async_copy(src_hbm.at[pl.ds(0, CH)], S[b], IS[b]).wait()
        pltpu.make_async_copy(dst_hbm.at[pl.ds(0, CH)], Dx[b], IS[b]).wait()

    def issue_gather(src_b, row_b):
        pltpu.async_copy(g_hbm.at[S[src_b]], R[row_b], GS[row_b])

    def wait_gather(b):
        pltpu.make_async_copy(g_hbm.at[S[b]], R[b], GS[b]).wait()

    def zbody(r, carry):
        for j in range(D // 16):
            r0[r, pl.ds(j * 16, 16)] = zv
        return carry

    lax.fori_loop(0, CH, zbody, 0)
    for j in range(ROWS_W // CH):
        pltpu.async_copy(r0, acc.at[pl.ds(s * ROWS_W + j * CH, CH), :], ws0)
    for j in range(ROWS_W // CH):
        pltpu.make_async_copy(
            r0, acc.at[pl.ds(s * ROWS_W, CH), :], ws0).wait()
    plsc.subcore_barrier()

    def step(i, b, tail):
        wait_gather(b)
        if tail >= 1:
            wait_idx(1 - b)
            issue_gather(1 - b, 1 - b)
        pltpu.sync_copy(R[b], acc.at[Dx[b]], add=True)
        if tail >= 2:
            issue_idx(i + 2, b)

    issue_idx(0, 0)
    issue_idx(1, 1)
    wait_idx(0)
    issue_gather(0, 0)

    def ebody(q, carry):
        i = q * 2
        step(i, 0, 2)
        step(i + 1, 1, 2)
        return carry

    lax.fori_loop(0, (LASTC - 2) // 2, ebody, 0)
    step(122, 0, 2)
    step(123, 1, 1)
    step(124, 0, 0)
    plsc.subcore_barrier()

    for j in range(ROWS_W // CH):
        b = j & 1
        r0_ = s * ROWS_W + j * CH
        if j >= 2:
            pltpu.make_async_copy(
                R[b], out_hbm.at[pl.ds(c * NP, CH), :], WS[b]).wait()
        pltpu.sync_copy(acc.at[pl.ds(r0_, CH), :], R[b])
        pltpu.async_copy(R[b], out_hbm.at[pl.ds(c * NP + r0_, CH), :], WS[b])
    for b in (0, 1):
        pltpu.make_async_copy(
            R[b], out_hbm.at[pl.ds(c * NP, CH), :], WS[b]).wait()


DEC_LAST = DEC_CHUNK - 1


@functools.partial(
    pl.kernel,
    out_type=jax.ShapeDtypeStruct((EDEC,), jnp.float32),
    mesh=_mesh,
    compiler_params=_SC_PARAMS,
    scratch_types=[
        pltpu.VMEM((CH,), jnp.int32), pltpu.VMEM((CH,), jnp.int32),
        pltpu.VMEM((CH,), jnp.int32), pltpu.VMEM((CH,), jnp.int32),
        pltpu.VMEM((CH, D), jnp.float32), pltpu.VMEM((CH, D), jnp.float32),
        pltpu.VMEM((CH, D), jnp.float32), pltpu.VMEM((CH, D), jnp.float32),
        pltpu.VMEM((DEC_PER_W,), jnp.float32),
        pltpu.SemaphoreType.DMA, pltpu.SemaphoreType.DMA,
        pltpu.SemaphoreType.DMA, pltpu.SemaphoreType.DMA,
        pltpu.SemaphoreType.DMA,
    ],
)
def _decoder_kernel(lat_hbm, ia_hbm, ib_hbm, out_hbm,
                    ia0, ia1, ib0, ib1, ra0, ra1, rb0, rb1, obuf,
                    is0, is1, gs0, gs1, os0):
    c = lax.axis_index("c")
    s = lax.axis_index("s")
    wid = s * NC + c
    lanes = lax.broadcasted_iota(jnp.int32, (16,), 0)
    IA = [ia0, ia1]
    IB = [ib0, ib1]
    RA = [ra0, ra1]
    RB = [rb0, rb1]
    IS = [is0, is1]
    GS = [gs0, gs1]

    def issue_idx(i, b):
        base = wid * DEC_PER_W + i * CH
        pltpu.async_copy(ia_hbm.at[pl.ds(base, CH)], IA[b], IS[b])
        pltpu.async_copy(ib_hbm.at[pl.ds(base, CH)], IB[b], IS[b])

    def wait_idx(b):
        pltpu.make_async_copy(ia_hbm.at[pl.ds(0, CH)], IA[b], IS[b]).wait()
        pltpu.make_async_copy(ib_hbm.at[pl.ds(0, CH)], IB[b], IS[b]).wait()

    def issue_gather(idx_b, row_b):
        pltpu.async_copy(lat_hbm.at[IA[idx_b]], RA[row_b], GS[row_b])
        pltpu.async_copy(lat_hbm.at[IB[idx_b]], RB[row_b], GS[row_b])

    def wait_gather(b):
        pltpu.make_async_copy(lat_hbm.at[IA[b]], RA[b], GS[b]).wait()
        pltpu.make_async_copy(lat_hbm.at[IB[b]], RB[b], GS[b]).wait()

    def compute(i, b):
        for g in range(CH // 16):
            evec = lanes + g * 16
            accs = [jnp.zeros((16,), jnp.float32) for _ in range(4)]
            for k in range(OUT):
                kvec = jnp.full((16,), k, jnp.int32)
                va = plsc.load_gather(RA[b], [evec, kvec])
                vb = plsc.load_gather(RB[b], [evec, kvec])
                accs[k & 3] = accs[k & 3] + va * vb
            acc = (accs[0] + accs[1]) + (accs[2] + accs[3])
            obuf[pl.ds(i * CH + g * 16, 16)] = acc

    def step(i, b, tail):
        wait_gather(b)
        if tail >= 1:
            wait_idx(1 - b)
            issue_gather(1 - b, 1 - b)
        compute(i, b)
        if tail >= 2:
            issue_idx(i + 2, b)

    issue_idx(0, 0)
    issue_idx(1, 1)
    wait_idx(0)
    issue_gather(0, 0)

    def ebody(q, carry):
        i = q * 2
        step(i, 0, 2)
        step(i + 1, 1, 2)
        return carry

    lax.fori_loop(0, (DEC_LAST - 1) // 2, ebody, 0)
    step(248, 0, 1)
    step(249, 1, 0)
    pltpu.sync_copy(obuf, out_hbm.at[pl.ds(wid * DEC_PER_W, DEC_PER_W)])


RB = 2048
GRID = NP // RB


def _dinv_block(degp):
    deg = degp[0] + degp[1] + 1.0
    return 1.0 / jnp.sqrt(deg)


def _tc1_body(x_ref, w1_ref, degp_ref, g1_ref):
    dinv = _dinv_block(degp_ref[...])
    h = jnp.dot(x_ref[...], w1_ref[...], preferred_element_type=jnp.float32)
    g1_ref[...] = h * dinv


def _tc2_body(p_ref, g1_ref, degp_ref, b1_ref, w2_ref, g2_ref):
    dinv = _dinv_block(degp_ref[...])
    agg = (p_ref[0] + p_ref[1] + g1_ref[...]) * dinv + b1_ref[...]
    h2 = jnp.maximum(agg, 0.0)
    t = jnp.dot(h2, w2_ref[...], preferred_element_type=jnp.float32) * dinv
    g2_ref[...] = jnp.concatenate(
        [t, jnp.zeros((RB, D - OUT), jnp.float32)], axis=1)


def _tc3_body(q_ref, g2_ref, degp_ref, b2_ref, lat_ref):
    dinv = _dinv_block(degp_ref[...])
    qsum = (q_ref[0] + q_ref[1] + g2_ref[...])[:, :OUT]
    lat = qsum * dinv + b2_ref[...]
    lat_ref[...] = jnp.concatenate(
        [lat, jnp.zeros((RB, D - OUT), jnp.float32)], axis=1)


def _row_spec(w):
    return pl.BlockSpec((RB, w), lambda i: (i, 0))


def _pair_spec(w):
    return pl.BlockSpec((2, RB, w), lambda i: (0, i, 0))


_DEGP_SPEC = pl.BlockSpec((2, RB, 1), lambda i: (0, i, 0))


def _full_spec(a, b):
    return pl.BlockSpec((a, b), lambda i: (0, 0))


_tc1 = pl.pallas_call(
    _tc1_body,
    grid=(GRID,),
    in_specs=[_row_spec(D), _full_spec(D, D), _DEGP_SPEC],
    out_specs=_row_spec(D),
    out_shape=jax.ShapeDtypeStruct((NP, D), jnp.float32),
)

_tc2 = pl.pallas_call(
    _tc2_body,
    grid=(GRID,),
    in_specs=[_pair_spec(D), _row_spec(D), _DEGP_SPEC,
              _full_spec(1, D), _full_spec(D, OUT)],
    out_specs=_row_spec(D),
    out_shape=jax.ShapeDtypeStruct((NP, D), jnp.float32),
)

_tc3 = pl.pallas_call(
    _tc3_body,
    grid=(GRID,),
    in_specs=[_pair_spec(D), _row_spec(D), _DEGP_SPEC,
              _full_spec(1, OUT)],
    out_specs=_row_spec(D),
    out_shape=jax.ShapeDtypeStruct((NP, D), jnp.float32),
)


def kernel(x, edge_index, pos_edge, neg_edge, W1, b1, W2, b2):
    src = edge_index[0]
    dst = edge_index[1]
    x_pad = jnp.pad(x, ((0, NP - N), (0, 0)))

    degp = _deg_kernel(dst).reshape(2, NP, 1)
    g1 = _tc1(x_pad, W1, degp)
    p = _scatter_kernel(g1, src, dst).reshape(2, NP, D)
    g2 = _tc2(p, g1, degp, b1.reshape(1, D), W2)
    q = _scatter_kernel(g2, src, dst).reshape(2, NP, D)
    latent = _tc3(q, g2, degp, b2.reshape(1, OUT))

    ia = jnp.concatenate([pos_edge[0], neg_edge[0]])
    ib = jnp.concatenate([pos_edge[1], neg_edge[1]])
    return _decoder_kernel(latent, ia, ib)
